# baseline (speedup 1.0000x reference)
import jax
import jax.numpy as jnp
from jax.experimental import pallas as pl
from jax.experimental.pallas import tpu as pltpu

N_KEYS = 100000
BITS = 512
ROWS = 5000
NB = N_KEYS // ROWS
CH = ROWS // 128
TAIL = ROWS - CH * 128


def _body(t_ref, w_ref, k_ref, o_ref):
    kb = k_ref[...].astype(jnp.bfloat16)
    d = jax.lax.dot_general(
        kb, w_ref[...], (((1,), (0,)), ((), ())),
        preferred_element_type=jnp.float32)
    d3 = d[:CH * 128].reshape(CH, 128, 128)
    row_i = jax.lax.broadcasted_iota(jnp.int32, (128, 128), 0)
    col_i = jax.lax.broadcasted_iota(jnp.int32, (128, 128), 1)
    eye = jnp.where(row_i == col_i, 1.0, 0.0)
    diag = jnp.sum(d3 * eye[None], axis=1)
    diag_tail = jnp.sum(d[CH * 128:] * eye[:TAIL], axis=0)
    t = t_ref[0]
    w_main = jnp.where(diag <= t, 1.0, 0.0).reshape(CH * 128)
    w_tail = jnp.where(diag_tail <= t, 1.0, 0.0)[:TAIL]
    o_ref[...] = jnp.concatenate([w_main, w_tail]).reshape(1, 1, ROWS)


def kernel(query, keys):
    q = jnp.reshape(query, (BITS,))
    w = (1.0 - 2.0 * q).astype(jnp.bfloat16)
    wmat = jnp.tile(w[:, None], (1, 128))
    t = (1.0 - jnp.sum(q)).reshape(1)
    return pl.pallas_call(
        _body,
        grid=(NB,),
        in_specs=[
            pl.BlockSpec(memory_space=pltpu.SMEM),
            pl.BlockSpec((BITS, 128), lambda i: (0, 0)),
            pl.BlockSpec((ROWS, BITS), lambda i: (i, 0)),
        ],
        out_specs=pl.BlockSpec((1, 1, ROWS), lambda i: (i, 0, 0)),
        out_shape=jax.ShapeDtypeStruct((NB, 1, ROWS), jnp.float32),
    )(t, wmat, keys).reshape(N_KEYS)
